# baseline (speedup 1.0000x reference)
import functools

import jax
import jax.numpy as jnp
from jax import lax
from jax.experimental import pallas as pl
from jax.experimental.pallas import tpu as pltpu
from jax.experimental.pallas import tpu_sc as plsc

TM1 = 128
TM = 64


def _dot(a, b):
    return lax.dot_general(a, b, (((1,), (1,)), ((), ())),
                           preferred_element_type=jnp.float32)


def _dot_fast(a, b):
    return lax.dot_general(a, b, (((1,), (1,)), ((), ())),
                           preferred_element_type=jnp.float32,
                           precision=lax.Precision.DEFAULT)


def _silu(v):
    return v * (1.0 / (1.0 + jnp.exp(-v)))


def _k1(x_ref, rt_ref, us_ref, gs_ref, ds_ref, idx_ref, ys_ref):
    xt = x_ref[...]
    logits = _dot(xt, rt_ref[...])
    mx = jnp.max(logits, axis=1, keepdims=True)
    col = lax.broadcasted_iota(jnp.int32, logits.shape, 1)
    idx = jnp.min(jnp.where(logits == mx, col, jnp.int32(2**30)), axis=1)
    idx_ref[0, 0, :] = idx.astype(jnp.int32)
    u = _dot_fast(xt, us_ref[...])
    g = _dot_fast(xt, gs_ref[...])
    h = _silu(g) * u
    ys_ref[...] = _dot_fast(h, ds_ref[...])


def _k2(g_r, t_r, rs_r, re_r,
        xs_ref, yss_ref, up_ref, gate_ref, down_ref, o_ref):
    w = pl.program_id(0)
    rs = rs_r[w]
    re = re_r[w]
    xt = xs_ref[...]
    u = _dot_fast(xt, up_ref[0])
    g = _dot_fast(xt, gate_ref[0])
    h = _silu(g) * u
    val = _dot_fast(h, down_ref[0]) + yss_ref[...]
    rows = lax.broadcasted_iota(jnp.int32, (TM, 1), 0)
    mask = (rows >= rs) & (rows < re)
    o_ref[...] = jnp.where(mask, val, o_ref[...])


def _schedule(idx, E, BT):
    ntiles = BT // TM
    W = ntiles + E - 1
    perm = jnp.argsort(idx).astype(jnp.int32)
    sizes = jnp.sum(idx[None, :] == jnp.arange(E, dtype=idx.dtype)[:, None],
                    axis=1).astype(jnp.int32)
    ends = jnp.cumsum(sizes)
    starts = ends - sizes
    t0 = starts // TM
    t1 = (ends + TM - 1) // TM
    u = jnp.where(sizes > 0, t1 - t0, 0)
    uend = jnp.cumsum(u)
    ustart = uend - u
    U = uend[E - 1]
    w = jnp.arange(W, dtype=jnp.int32)
    gw = jnp.clip(jnp.searchsorted(uend, w, side="right"), 0, E - 1)
    gw = gw.astype(jnp.int32)
    tile = t0[gw] + (w - ustart[gw])
    rs = jnp.maximum(starts[gw] - tile * TM, 0)
    re = jnp.minimum(ends[gw] - tile * TM, TM)
    valid = w < U
    last_g = jnp.max(jnp.where(sizes > 0, jnp.arange(E, dtype=jnp.int32), -1))
    gw = jnp.where(valid, gw, last_g).astype(jnp.int32)
    tile = jnp.where(valid, tile, ntiles - 1).astype(jnp.int32)
    rs = jnp.where(valid, rs, 0).astype(jnp.int32)
    re = jnp.where(valid, re, 0).astype(jnp.int32)
    return gw, tile, rs, re, perm


_NW = 32
_RPW = 2048 // _NW


def _sc_gather_body(x_hbm, ys_hbm, perm_hbm, xs_out, yss_out,
                    idx_v, rows_v, sem):
    wid = lax.axis_index("s") * 2 + lax.axis_index("c")
    base = wid * _RPW
    pltpu.sync_copy(perm_hbm.at[pl.ds(base, _RPW)], idx_v)
    pltpu.async_copy(x_hbm.at[idx_v], rows_v, sem).wait()
    pltpu.sync_copy(rows_v, xs_out.at[pl.ds(base, _RPW)])
    pltpu.async_copy(ys_hbm.at[idx_v], rows_v, sem).wait()
    pltpu.sync_copy(rows_v, yss_out.at[pl.ds(base, _RPW)])


def _sc_scatter_body(ysort_hbm, perm_hbm, y_out, idx_v, rows_v, sem):
    wid = lax.axis_index("s") * 2 + lax.axis_index("c")
    base = wid * _RPW
    pltpu.sync_copy(perm_hbm.at[pl.ds(base, _RPW)], idx_v)
    pltpu.sync_copy(ysort_hbm.at[pl.ds(base, _RPW)], rows_v)
    pltpu.async_copy(rows_v, y_out.at[idx_v], sem).wait()


def kernel(x, up, gate, down, router, up_s, gate_s, down_s):
    b, t, c = x.shape
    BT = b * t
    E, H, C = up.shape
    x2 = x.reshape(BT, c)

    idx3, ys = pl.pallas_call(
        _k1,
        grid=(BT // TM1,),
        in_specs=[
            pl.BlockSpec((TM1, C), lambda i: (i, 0)),
            pl.BlockSpec((E, C), lambda i: (0, 0)),
            pl.BlockSpec((H, C), lambda i: (0, 0)),
            pl.BlockSpec((H, C), lambda i: (0, 0)),
            pl.BlockSpec((C, H), lambda i: (0, 0)),
        ],
        out_specs=[
            pl.BlockSpec((1, 1, TM1), lambda i: (i, 0, 0)),
            pl.BlockSpec((TM1, C), lambda i: (i, 0)),
        ],
        out_shape=[
            jax.ShapeDtypeStruct((BT // TM1, 1, TM1), jnp.int32),
            jax.ShapeDtypeStruct((BT, C), jnp.float32),
        ],
    )(x2, router, up_s, gate_s, down_s)
    idx = idx3.reshape(BT)

    gw, tile, rs, re, perm = _schedule(idx, E, BT)
    W = BT // TM + E - 1

    sc_mesh = plsc.VectorSubcoreMesh(core_axis_name="c", subcore_axis_name="s")
    sc_gather = functools.partial(
        pl.kernel,
        mesh=sc_mesh,
        out_type=[
            jax.ShapeDtypeStruct((BT, C), jnp.float32),
            jax.ShapeDtypeStruct((BT, C), jnp.float32),
        ],
        scratch_types=[
            pltpu.VMEM((_RPW,), jnp.int32),
            pltpu.VMEM((_RPW, C), jnp.float32),
            pltpu.SemaphoreType.DMA,
        ],
    )(_sc_gather_body)
    xs, yss = sc_gather(x2, ys, perm)

    y_sorted = pl.pallas_call(
        _k2,
        grid_spec=pltpu.PrefetchScalarGridSpec(
            num_scalar_prefetch=4,
            grid=(W,),
            in_specs=[
                pl.BlockSpec((TM, C), lambda w, g, t_, r1, r2: (t_[w], 0)),
                pl.BlockSpec((TM, C), lambda w, g, t_, r1, r2: (t_[w], 0)),
                pl.BlockSpec((1, H, C), lambda w, g, t_, r1, r2: (g[w], 0, 0)),
                pl.BlockSpec((1, H, C), lambda w, g, t_, r1, r2: (g[w], 0, 0)),
                pl.BlockSpec((1, C, H), lambda w, g, t_, r1, r2: (g[w], 0, 0)),
            ],
            out_specs=pl.BlockSpec((TM, C), lambda w, g, t_, r1, r2: (t_[w], 0)),
        ),
        out_shape=jax.ShapeDtypeStruct((BT, C), jnp.float32),
    )(gw, tile, rs, re, xs, yss, up, gate, down)

    sc_scatter = functools.partial(
        pl.kernel,
        mesh=sc_mesh,
        out_type=jax.ShapeDtypeStruct((BT, C), jnp.float32),
        scratch_types=[
            pltpu.VMEM((_RPW,), jnp.int32),
            pltpu.VMEM((_RPW, C), jnp.float32),
            pltpu.SemaphoreType.DMA,
        ],
    )(_sc_scatter_body)
    y2 = sc_scatter(y_sorted, perm)

    return y2.reshape(b, t, c)


# speedup vs baseline: 1.3632x; 1.3632x over previous
import functools

import jax
import jax.numpy as jnp
from jax import lax
from jax.experimental import pallas as pl
from jax.experimental.pallas import tpu as pltpu
from jax.experimental.pallas import tpu_sc as plsc

TM1 = 128
TM = 256


def _dot(a, b):
    return lax.dot_general(a, b, (((1,), (1,)), ((), ())),
                           preferred_element_type=jnp.float32)


def _dot_fast(a, b):
    return lax.dot_general(a, b, (((1,), (1,)), ((), ())),
                           preferred_element_type=jnp.float32,
                           precision=lax.Precision.DEFAULT)


def _silu(v):
    return v * (1.0 / (1.0 + jnp.exp(-v)))


def _k1(x_ref, rt_ref, us_ref, gs_ref, ds_ref, idx_ref, ys_ref):
    xt = x_ref[...]
    logits = _dot(xt, rt_ref[...])
    mx = jnp.max(logits, axis=1, keepdims=True)
    col = lax.broadcasted_iota(jnp.int32, logits.shape, 1)
    idx = jnp.min(jnp.where(logits == mx, col, jnp.int32(2**30)), axis=1)
    idx_ref[0, 0, :] = idx.astype(jnp.int32)
    u = _dot_fast(xt, us_ref[...])
    g = _dot_fast(xt, gs_ref[...])
    h = _silu(g) * u
    ys_ref[...] = _dot_fast(h, ds_ref[...])


def _k2(g_r, t_r, rs_r, re_r,
        xs_ref, yss_ref, up_ref, gate_ref, down_ref, o_ref):
    w = pl.program_id(0)
    rs = rs_r[w]
    re = re_r[w]
    xt = xs_ref[...]
    u = _dot_fast(xt, up_ref[0])
    g = _dot_fast(xt, gate_ref[0])
    h = _silu(g) * u
    val = _dot_fast(h, down_ref[0]) + yss_ref[...]
    rows = lax.broadcasted_iota(jnp.int32, (TM, 1), 0)
    mask = (rows >= rs) & (rows < re)
    o_ref[...] = jnp.where(mask, val, o_ref[...])


def _schedule(idx, E, BT):
    ntiles = BT // TM
    W = ntiles + E - 1
    perm = jnp.argsort(idx).astype(jnp.int32)
    sizes = jnp.sum(idx[None, :] == jnp.arange(E, dtype=idx.dtype)[:, None],
                    axis=1).astype(jnp.int32)
    ends = jnp.cumsum(sizes)
    starts = ends - sizes
    t0 = starts // TM
    t1 = (ends + TM - 1) // TM
    u = jnp.where(sizes > 0, t1 - t0, 0)
    uend = jnp.cumsum(u)
    ustart = uend - u
    U = uend[E - 1]
    w = jnp.arange(W, dtype=jnp.int32)
    gw = jnp.clip(jnp.searchsorted(uend, w, side="right"), 0, E - 1)
    gw = gw.astype(jnp.int32)
    tile = t0[gw] + (w - ustart[gw])
    rs = jnp.maximum(starts[gw] - tile * TM, 0)
    re = jnp.minimum(ends[gw] - tile * TM, TM)
    valid = w < U
    last_g = jnp.max(jnp.where(sizes > 0, jnp.arange(E, dtype=jnp.int32), -1))
    gw = jnp.where(valid, gw, last_g).astype(jnp.int32)
    tile = jnp.where(valid, tile, ntiles - 1).astype(jnp.int32)
    rs = jnp.where(valid, rs, 0).astype(jnp.int32)
    re = jnp.where(valid, re, 0).astype(jnp.int32)
    return gw, tile, rs, re, perm


_NW = 32
_RPW = 2048 // _NW


def _sc_gather_body(x_hbm, ys_hbm, perm_hbm, xs_out, yss_out,
                    idx_v, rows_v, sem):
    wid = lax.axis_index("s") * 2 + lax.axis_index("c")
    base = wid * _RPW
    pltpu.sync_copy(perm_hbm.at[pl.ds(base, _RPW)], idx_v)
    pltpu.async_copy(x_hbm.at[idx_v], rows_v, sem).wait()
    pltpu.sync_copy(rows_v, xs_out.at[pl.ds(base, _RPW)])
    pltpu.async_copy(ys_hbm.at[idx_v], rows_v, sem).wait()
    pltpu.sync_copy(rows_v, yss_out.at[pl.ds(base, _RPW)])


def _sc_scatter_body(ysort_hbm, perm_hbm, y_out, idx_v, rows_v, sem):
    wid = lax.axis_index("s") * 2 + lax.axis_index("c")
    base = wid * _RPW
    pltpu.sync_copy(perm_hbm.at[pl.ds(base, _RPW)], idx_v)
    pltpu.sync_copy(ysort_hbm.at[pl.ds(base, _RPW)], rows_v)
    pltpu.async_copy(rows_v, y_out.at[idx_v], sem).wait()


def kernel(x, up, gate, down, router, up_s, gate_s, down_s):
    b, t, c = x.shape
    BT = b * t
    E, H, C = up.shape
    x2 = x.reshape(BT, c)

    idx3, ys = pl.pallas_call(
        _k1,
        grid=(BT // TM1,),
        in_specs=[
            pl.BlockSpec((TM1, C), lambda i: (i, 0)),
            pl.BlockSpec((E, C), lambda i: (0, 0)),
            pl.BlockSpec((H, C), lambda i: (0, 0)),
            pl.BlockSpec((H, C), lambda i: (0, 0)),
            pl.BlockSpec((C, H), lambda i: (0, 0)),
        ],
        out_specs=[
            pl.BlockSpec((1, 1, TM1), lambda i: (i, 0, 0)),
            pl.BlockSpec((TM1, C), lambda i: (i, 0)),
        ],
        out_shape=[
            jax.ShapeDtypeStruct((BT // TM1, 1, TM1), jnp.int32),
            jax.ShapeDtypeStruct((BT, C), jnp.float32),
        ],
    )(x2, router, up_s, gate_s, down_s)
    idx = idx3.reshape(BT)

    gw, tile, rs, re, perm = _schedule(idx, E, BT)
    W = BT // TM + E - 1

    sc_mesh = plsc.VectorSubcoreMesh(core_axis_name="c", subcore_axis_name="s")
    sc_gather = functools.partial(
        pl.kernel,
        mesh=sc_mesh,
        out_type=[
            jax.ShapeDtypeStruct((BT, C), jnp.float32),
            jax.ShapeDtypeStruct((BT, C), jnp.float32),
        ],
        scratch_types=[
            pltpu.VMEM((_RPW,), jnp.int32),
            pltpu.VMEM((_RPW, C), jnp.float32),
            pltpu.SemaphoreType.DMA,
        ],
    )(_sc_gather_body)
    xs, yss = sc_gather(x2, ys, perm)

    y_sorted = pl.pallas_call(
        _k2,
        grid_spec=pltpu.PrefetchScalarGridSpec(
            num_scalar_prefetch=4,
            grid=(W,),
            in_specs=[
                pl.BlockSpec((TM, C), lambda w, g, t_, r1, r2: (t_[w], 0)),
                pl.BlockSpec((TM, C), lambda w, g, t_, r1, r2: (t_[w], 0)),
                pl.BlockSpec((1, H, C), lambda w, g, t_, r1, r2: (g[w], 0, 0)),
                pl.BlockSpec((1, H, C), lambda w, g, t_, r1, r2: (g[w], 0, 0)),
                pl.BlockSpec((1, C, H), lambda w, g, t_, r1, r2: (g[w], 0, 0)),
            ],
            out_specs=pl.BlockSpec((TM, C), lambda w, g, t_, r1, r2: (t_[w], 0)),
        ),
        out_shape=jax.ShapeDtypeStruct((BT, C), jnp.float32),
    )(gw, tile, rs, re, xs, yss, up, gate, down)

    sc_scatter = functools.partial(
        pl.kernel,
        mesh=sc_mesh,
        out_type=jax.ShapeDtypeStruct((BT, C), jnp.float32),
        scratch_types=[
            pltpu.VMEM((_RPW,), jnp.int32),
            pltpu.VMEM((_RPW, C), jnp.float32),
            pltpu.SemaphoreType.DMA,
        ],
    )(_sc_scatter_body)
    y2 = sc_scatter(y_sorted, perm)

    return y2.reshape(b, t, c)


# speedup vs baseline: 1.5338x; 1.1252x over previous
import functools

import jax
import jax.numpy as jnp
from jax import lax
from jax.experimental import pallas as pl
from jax.experimental.pallas import tpu as pltpu
from jax.experimental.pallas import tpu_sc as plsc

TM1 = 256
TM = 256


def _dot(a, b):
    return lax.dot_general(a, b, (((1,), (1,)), ((), ())),
                           preferred_element_type=jnp.float32)


def _dot_fast(a, b):
    return lax.dot_general(a, b, (((1,), (1,)), ((), ())),
                           preferred_element_type=jnp.float32,
                           precision=lax.Precision.DEFAULT)


def _silu(v):
    return v * (1.0 / (1.0 + jnp.exp(-v)))


def _k1(x_ref, rt_ref, us_ref, gs_ref, ds_ref, idx_ref, ys_ref):
    xt = x_ref[...]
    logits = _dot(xt, rt_ref[...])
    mx = jnp.max(logits, axis=1, keepdims=True)
    col = lax.broadcasted_iota(jnp.int32, logits.shape, 1)
    idx = jnp.min(jnp.where(logits == mx, col, jnp.int32(2**30)), axis=1)
    idx_ref[0, 0, :] = idx.astype(jnp.int32)
    u = _dot_fast(xt, us_ref[...])
    g = _dot_fast(xt, gs_ref[...])
    h = _silu(g) * u
    ys_ref[...] = _dot_fast(h, ds_ref[...])


def _k2(g_r, t_r, rs_r, re_r,
        xs_ref, yss_ref, up_ref, gate_ref, down_ref, o_ref):
    w = pl.program_id(0)
    rs = rs_r[w]
    re = re_r[w]
    xt = xs_ref[...]
    u = _dot_fast(xt, up_ref[0])
    g = _dot_fast(xt, gate_ref[0])
    h = _silu(g) * u
    val = _dot_fast(h, down_ref[0]) + yss_ref[...]
    rows = lax.broadcasted_iota(jnp.int32, (TM, 1), 0)
    mask = (rows >= rs) & (rows < re)
    o_ref[...] = jnp.where(mask, val, o_ref[...])


def _schedule(idx, E, BT):
    ntiles = BT // TM
    W = ntiles + E - 1
    perm = jnp.argsort(idx).astype(jnp.int32)
    sizes = jnp.sum(idx[None, :] == jnp.arange(E, dtype=idx.dtype)[:, None],
                    axis=1).astype(jnp.int32)
    ends = jnp.cumsum(sizes)
    starts = ends - sizes
    t0 = starts // TM
    t1 = (ends + TM - 1) // TM
    u = jnp.where(sizes > 0, t1 - t0, 0)
    uend = jnp.cumsum(u)
    ustart = uend - u
    U = uend[E - 1]
    w = jnp.arange(W, dtype=jnp.int32)
    gw = jnp.clip(jnp.searchsorted(uend, w, side="right"), 0, E - 1)
    gw = gw.astype(jnp.int32)
    tile = t0[gw] + (w - ustart[gw])
    rs = jnp.maximum(starts[gw] - tile * TM, 0)
    re = jnp.minimum(ends[gw] - tile * TM, TM)
    valid = w < U
    last_g = jnp.max(jnp.where(sizes > 0, jnp.arange(E, dtype=jnp.int32), -1))
    gw = jnp.where(valid, gw, last_g).astype(jnp.int32)
    tile = jnp.where(valid, tile, ntiles - 1).astype(jnp.int32)
    rs = jnp.where(valid, rs, 0).astype(jnp.int32)
    re = jnp.where(valid, re, 0).astype(jnp.int32)
    return gw, tile, rs, re, perm


_NW = 32
_RPW = 2048 // _NW


def _sc_gather_body(x_hbm, ys_hbm, perm_hbm, xs_out, yss_out,
                    idx_v, rows_v, sem):
    wid = lax.axis_index("s") * 2 + lax.axis_index("c")
    base = wid * _RPW
    pltpu.sync_copy(perm_hbm.at[pl.ds(base, _RPW)], idx_v)
    pltpu.async_copy(x_hbm.at[idx_v], rows_v, sem).wait()
    pltpu.sync_copy(rows_v, xs_out.at[pl.ds(base, _RPW)])
    pltpu.async_copy(ys_hbm.at[idx_v], rows_v, sem).wait()
    pltpu.sync_copy(rows_v, yss_out.at[pl.ds(base, _RPW)])


def _sc_scatter_body(ysort_hbm, perm_hbm, y_out, idx_v, rows_v, sem):
    wid = lax.axis_index("s") * 2 + lax.axis_index("c")
    base = wid * _RPW
    pltpu.sync_copy(perm_hbm.at[pl.ds(base, _RPW)], idx_v)
    pltpu.sync_copy(ysort_hbm.at[pl.ds(base, _RPW)], rows_v)
    pltpu.async_copy(rows_v, y_out.at[idx_v], sem).wait()


def kernel(x, up, gate, down, router, up_s, gate_s, down_s):
    b, t, c = x.shape
    BT = b * t
    E, H, C = up.shape
    x2 = x.reshape(BT, c)

    idx3, ys = pl.pallas_call(
        _k1,
        grid=(BT // TM1,),
        in_specs=[
            pl.BlockSpec((TM1, C), lambda i: (i, 0)),
            pl.BlockSpec((E, C), lambda i: (0, 0)),
            pl.BlockSpec((H, C), lambda i: (0, 0)),
            pl.BlockSpec((H, C), lambda i: (0, 0)),
            pl.BlockSpec((C, H), lambda i: (0, 0)),
        ],
        out_specs=[
            pl.BlockSpec((1, 1, TM1), lambda i: (i, 0, 0)),
            pl.BlockSpec((TM1, C), lambda i: (i, 0)),
        ],
        out_shape=[
            jax.ShapeDtypeStruct((BT // TM1, 1, TM1), jnp.int32),
            jax.ShapeDtypeStruct((BT, C), jnp.float32),
        ],
    )(x2, router, up_s, gate_s, down_s)
    idx = idx3.reshape(BT)

    gw, tile, rs, re, perm = _schedule(idx, E, BT)
    W = BT // TM + E - 1

    sc_mesh = plsc.VectorSubcoreMesh(core_axis_name="c", subcore_axis_name="s")
    sc_gather = functools.partial(
        pl.kernel,
        mesh=sc_mesh,
        out_type=[
            jax.ShapeDtypeStruct((BT, C), jnp.float32),
            jax.ShapeDtypeStruct((BT, C), jnp.float32),
        ],
        scratch_types=[
            pltpu.VMEM((_RPW,), jnp.int32),
            pltpu.VMEM((_RPW, C), jnp.float32),
            pltpu.SemaphoreType.DMA,
        ],
    )(_sc_gather_body)
    xs, yss = sc_gather(x2, ys, perm)

    y_sorted = pl.pallas_call(
        _k2,
        grid_spec=pltpu.PrefetchScalarGridSpec(
            num_scalar_prefetch=4,
            grid=(W,),
            in_specs=[
                pl.BlockSpec((TM, C), lambda w, g, t_, r1, r2: (t_[w], 0)),
                pl.BlockSpec((TM, C), lambda w, g, t_, r1, r2: (t_[w], 0)),
                pl.BlockSpec((1, H, C), lambda w, g, t_, r1, r2: (g[w], 0, 0)),
                pl.BlockSpec((1, H, C), lambda w, g, t_, r1, r2: (g[w], 0, 0)),
                pl.BlockSpec((1, C, H), lambda w, g, t_, r1, r2: (g[w], 0, 0)),
            ],
            out_specs=pl.BlockSpec((TM, C), lambda w, g, t_, r1, r2: (t_[w], 0)),
        ),
        out_shape=jax.ShapeDtypeStruct((BT, C), jnp.float32),
    )(gw, tile, rs, re, xs, yss, up, gate, down)

    sc_scatter = functools.partial(
        pl.kernel,
        mesh=sc_mesh,
        out_type=jax.ShapeDtypeStruct((BT, C), jnp.float32),
        scratch_types=[
            pltpu.VMEM((_RPW,), jnp.int32),
            pltpu.VMEM((_RPW, C), jnp.float32),
            pltpu.SemaphoreType.DMA,
        ],
    )(_sc_scatter_body)
    y2 = sc_scatter(y_sorted, perm)

    return y2.reshape(b, t, c)


# speedup vs baseline: 1.5403x; 1.0042x over previous
import functools

import jax
import jax.numpy as jnp
from jax import lax
from jax.experimental import pallas as pl
from jax.experimental.pallas import tpu as pltpu
from jax.experimental.pallas import tpu_sc as plsc

TM1 = 512
TM = 256


def _dot(a, b):
    return lax.dot_general(a, b, (((1,), (1,)), ((), ())),
                           preferred_element_type=jnp.float32)


def _dot_fast(a, b):
    return lax.dot_general(a, b, (((1,), (1,)), ((), ())),
                           preferred_element_type=jnp.float32,
                           precision=lax.Precision.DEFAULT)


def _silu(v):
    return v * (1.0 / (1.0 + jnp.exp(-v)))


def _k1(x_ref, rt_ref, us_ref, gs_ref, ds_ref, idx_ref, ys_ref):
    xt = x_ref[...]
    logits = _dot(xt, rt_ref[...])
    mx = jnp.max(logits, axis=1, keepdims=True)
    col = lax.broadcasted_iota(jnp.int32, logits.shape, 1)
    idx = jnp.min(jnp.where(logits == mx, col, jnp.int32(2**30)), axis=1)
    idx_ref[0, 0, :] = idx.astype(jnp.int32)
    u = _dot_fast(xt, us_ref[...])
    g = _dot_fast(xt, gs_ref[...])
    h = _silu(g) * u
    ys_ref[...] = _dot_fast(h, ds_ref[...])


def _k2(g_r, t_r, rs_r, re_r,
        xs_ref, yss_ref, up_ref, gate_ref, down_ref, o_ref):
    w = pl.program_id(0)
    rs = rs_r[w]
    re = re_r[w]
    xt = xs_ref[...]
    u = _dot_fast(xt, up_ref[0])
    g = _dot_fast(xt, gate_ref[0])
    h = _silu(g) * u
    val = _dot_fast(h, down_ref[0]) + yss_ref[...]
    rows = lax.broadcasted_iota(jnp.int32, (TM, 1), 0)
    mask = (rows >= rs) & (rows < re)
    o_ref[...] = jnp.where(mask, val, o_ref[...])


def _schedule(idx, E, BT):
    ntiles = BT // TM
    W = ntiles + E - 1
    perm = jnp.argsort(idx).astype(jnp.int32)
    sizes = jnp.sum(idx[None, :] == jnp.arange(E, dtype=idx.dtype)[:, None],
                    axis=1).astype(jnp.int32)
    ends = jnp.cumsum(sizes)
    starts = ends - sizes
    t0 = starts // TM
    t1 = (ends + TM - 1) // TM
    u = jnp.where(sizes > 0, t1 - t0, 0)
    uend = jnp.cumsum(u)
    ustart = uend - u
    U = uend[E - 1]
    w = jnp.arange(W, dtype=jnp.int32)
    gw = jnp.clip(jnp.searchsorted(uend, w, side="right"), 0, E - 1)
    gw = gw.astype(jnp.int32)
    tile = t0[gw] + (w - ustart[gw])
    rs = jnp.maximum(starts[gw] - tile * TM, 0)
    re = jnp.minimum(ends[gw] - tile * TM, TM)
    valid = w < U
    last_g = jnp.max(jnp.where(sizes > 0, jnp.arange(E, dtype=jnp.int32), -1))
    gw = jnp.where(valid, gw, last_g).astype(jnp.int32)
    tile = jnp.where(valid, tile, ntiles - 1).astype(jnp.int32)
    rs = jnp.where(valid, rs, 0).astype(jnp.int32)
    re = jnp.where(valid, re, 0).astype(jnp.int32)
    return gw, tile, rs, re, perm


_NW = 32
_RPW = 2048 // _NW


def _sc_gather_body(x_hbm, ys_hbm, perm_hbm, xs_out, yss_out,
                    idx_v, rows_v, sem):
    wid = lax.axis_index("s") * 2 + lax.axis_index("c")
    base = wid * _RPW
    pltpu.sync_copy(perm_hbm.at[pl.ds(base, _RPW)], idx_v)
    pltpu.async_copy(x_hbm.at[idx_v], rows_v, sem).wait()
    pltpu.sync_copy(rows_v, xs_out.at[pl.ds(base, _RPW)])
    pltpu.async_copy(ys_hbm.at[idx_v], rows_v, sem).wait()
    pltpu.sync_copy(rows_v, yss_out.at[pl.ds(base, _RPW)])


def _sc_scatter_body(ysort_hbm, perm_hbm, y_out, idx_v, rows_v, sem):
    wid = lax.axis_index("s") * 2 + lax.axis_index("c")
    base = wid * _RPW
    pltpu.sync_copy(perm_hbm.at[pl.ds(base, _RPW)], idx_v)
    pltpu.sync_copy(ysort_hbm.at[pl.ds(base, _RPW)], rows_v)
    pltpu.async_copy(rows_v, y_out.at[idx_v], sem).wait()


def kernel(x, up, gate, down, router, up_s, gate_s, down_s):
    b, t, c = x.shape
    BT = b * t
    E, H, C = up.shape
    x2 = x.reshape(BT, c)

    idx3, ys = pl.pallas_call(
        _k1,
        grid=(BT // TM1,),
        in_specs=[
            pl.BlockSpec((TM1, C), lambda i: (i, 0)),
            pl.BlockSpec((E, C), lambda i: (0, 0)),
            pl.BlockSpec((H, C), lambda i: (0, 0)),
            pl.BlockSpec((H, C), lambda i: (0, 0)),
            pl.BlockSpec((C, H), lambda i: (0, 0)),
        ],
        out_specs=[
            pl.BlockSpec((1, 1, TM1), lambda i: (i, 0, 0)),
            pl.BlockSpec((TM1, C), lambda i: (i, 0)),
        ],
        out_shape=[
            jax.ShapeDtypeStruct((BT // TM1, 1, TM1), jnp.int32),
            jax.ShapeDtypeStruct((BT, C), jnp.float32),
        ],
    )(x2, router, up_s, gate_s, down_s)
    idx = idx3.reshape(BT)

    gw, tile, rs, re, perm = _schedule(idx, E, BT)
    W = BT // TM + E - 1

    sc_mesh = plsc.VectorSubcoreMesh(core_axis_name="c", subcore_axis_name="s")
    sc_gather = functools.partial(
        pl.kernel,
        mesh=sc_mesh,
        out_type=[
            jax.ShapeDtypeStruct((BT, C), jnp.float32),
            jax.ShapeDtypeStruct((BT, C), jnp.float32),
        ],
        scratch_types=[
            pltpu.VMEM((_RPW,), jnp.int32),
            pltpu.VMEM((_RPW, C), jnp.float32),
            pltpu.SemaphoreType.DMA,
        ],
    )(_sc_gather_body)
    xs, yss = sc_gather(x2, ys, perm)

    y_sorted = pl.pallas_call(
        _k2,
        grid_spec=pltpu.PrefetchScalarGridSpec(
            num_scalar_prefetch=4,
            grid=(W,),
            in_specs=[
                pl.BlockSpec((TM, C), lambda w, g, t_, r1, r2: (t_[w], 0)),
                pl.BlockSpec((TM, C), lambda w, g, t_, r1, r2: (t_[w], 0)),
                pl.BlockSpec((1, H, C), lambda w, g, t_, r1, r2: (g[w], 0, 0)),
                pl.BlockSpec((1, H, C), lambda w, g, t_, r1, r2: (g[w], 0, 0)),
                pl.BlockSpec((1, C, H), lambda w, g, t_, r1, r2: (g[w], 0, 0)),
            ],
            out_specs=pl.BlockSpec((TM, C), lambda w, g, t_, r1, r2: (t_[w], 0)),
        ),
        out_shape=jax.ShapeDtypeStruct((BT, C), jnp.float32),
    )(gw, tile, rs, re, xs, yss, up, gate, down)

    sc_scatter = functools.partial(
        pl.kernel,
        mesh=sc_mesh,
        out_type=jax.ShapeDtypeStruct((BT, C), jnp.float32),
        scratch_types=[
            pltpu.VMEM((_RPW,), jnp.int32),
            pltpu.VMEM((_RPW, C), jnp.float32),
            pltpu.SemaphoreType.DMA,
        ],
    )(_sc_scatter_body)
    y2 = sc_scatter(y_sorted, perm)

    return y2.reshape(b, t, c)
